# baseline (speedup 1.0000x reference)
import jax
import jax.numpy as jnp
from jax import lax
from jax.experimental import pallas as pl
from jax.experimental.pallas import tpu as pltpu

MAX_OUT = 100
IOU_THR = 0.7
OUT_SIZE = 24
NEG = float("-inf")
G = 10
NCHUNK = MAX_OUT // G


def _body(rects_ref, coordt_ref, img2_ref, sel_ref, crops_ref, bb_ref,
          sm_ref, area_ref, imgs_ref):
    B = rects_ref.shape[0]
    R8, CN = rects_ref.shape[2], rects_ref.shape[3]
    C, H, W = imgs_ref.shape[1], imgs_ref.shape[2], imgs_ref.shape[3]

    flat = (lax.broadcasted_iota(jnp.int32, (R8, CN), 0) * CN
            + lax.broadcasted_iota(jnp.int32, (R8, CN), 1))
    bigint = jnp.int32(2 ** 30)
    subl8 = lax.broadcasted_iota(jnp.int32, (8, 8), 0)
    lane8g = lax.broadcasted_iota(jnp.int32, (8, 8), 1)
    lane8 = lax.broadcasted_iota(jnp.int32, (1, 8), 1)

    for b in range(B):
        for c in range(C):
            imgs_ref[b, c] = lax.dot_general(
                img2_ref[b], sel_ref[c], (((1,), (0,)), ((), ())),
                preferred_element_type=jnp.float32).astype(jnp.bfloat16)

    for b in range(B):
        area_ref[b] = ((rects_ref[b, 2] - rects_ref[b, 0])
                       * (rects_ref[b, 3] - rects_ref[b, 1]))

    def step(i, carry):
        ms = [jnp.max(carry[b]) for b in range(B)]
        idxs = [jnp.min(jnp.where(carry[b] == ms[b], flat, bigint))
                for b in range(B)]
        outs = []
        for b in range(B):
            m, idxsel = ms[b], idxs[b]
            valid = m > NEG
            cc = idxsel % CN
            rr = idxsel // CN
            ca = pl.multiple_of((cc // 8) * 8, 8)
            ss = cc - ca
            pick = (subl8 == ss) & (lane8g == rr)
            bx1 = jnp.sum(jnp.where(pick, coordt_ref[b, 0, pl.ds(ca, 8), :], 0.0))
            by1 = jnp.sum(jnp.where(pick, coordt_ref[b, 1, pl.ds(ca, 8), :], 0.0))
            bx2 = jnp.sum(jnp.where(pick, coordt_ref[b, 2, pl.ds(ca, 8), :], 0.0))
            by2 = jnp.sum(jnp.where(pick, coordt_ref[b, 3, pl.ds(ca, 8), :], 0.0))

            x1a = rects_ref[b, 0]
            y1a = rects_ref[b, 1]
            x2a = rects_ref[b, 2]
            y2a = rects_ref[b, 3]
            ix1 = jnp.maximum(bx1, x1a)
            iy1 = jnp.maximum(by1, y1a)
            ix2 = jnp.minimum(bx2, x2a)
            iy2 = jnp.minimum(by2, y2a)
            inter = (jnp.maximum(ix2 - ix1, 0.0)
                     * jnp.maximum(iy2 - iy1, 0.0))
            area_b = (bx2 - bx1) * (by2 - by1)
            iou = inter / (area_b + area_ref[b] - inter)
            supp = (iou > IOU_THR) & valid
            outs.append(jnp.where(supp, NEG, carry[b]))

            vx1 = jnp.where(valid, bx1, 0.0)
            vy1 = jnp.where(valid, by1, 0.0)
            vx2 = jnp.where(valid, bx2, 0.0)
            vy2 = jnp.where(valid, by2, 0.0)
            vm = jnp.where(valid, m, 0.0)
            row = (jnp.where(lane8 == 0, vx1, 0.0)
                   + jnp.where(lane8 == 1, vy1, 0.0)
                   + jnp.where(lane8 == 2, vx2, 0.0)
                   + jnp.where(lane8 == 3, vy2, 0.0)
                   + jnp.where(lane8 == 4, vm, 0.0))
            bb_ref[b, pl.ds(i, 1), :] = row
            sm_ref[b, i, 0] = vx1
            sm_ref[b, i, 1] = vy1
            sm_ref[b, i, 2] = vx2
            sm_ref[b, i, 3] = vy2
        return tuple(outs)

    lax.fori_loop(0, MAX_OUT, step,
                  tuple(rects_ref[b, 4] for b in range(B)))

    iic = lax.broadcasted_iota(jnp.int32, (OUT_SIZE, 1), 0).astype(jnp.float32)
    jjf = lax.broadcasted_iota(jnp.int32, (G * OUT_SIZE, H), 1).astype(jnp.float32)
    kkf = lax.broadcasted_iota(jnp.int32, (G * OUT_SIZE, W), 1).astype(jnp.float32)

    def chunk(k, _):
        for b in range(B):
            sy_l, sx_l = [], []
            for g in range(G):
                idx = k * G + g
                x1q = sm_ref[b, idx, 0].astype(jnp.int32)
                y1q = sm_ref[b, idx, 1].astype(jnp.int32)
                x2q = sm_ref[b, idx, 2].astype(jnp.int32)
                y2q = sm_ref[b, idx, 3].astype(jnp.int32)
                h = (y2q - y1q).astype(jnp.float32)
                w = (x2q - x1q).astype(jnp.float32)
                y0 = (y1q - 1).astype(jnp.float32)
                x0 = (x1q - 1).astype(jnp.float32)
                sy_l.append(jnp.maximum(y0 + iic * h / OUT_SIZE, 0.0))
                sx_l.append(jnp.maximum(x0 + iic * w / OUT_SIZE, 0.0))
            sycol = jnp.concatenate(sy_l, axis=0)
            sxcol = jnp.concatenate(sx_l, axis=0)
            ry = jnp.maximum(1.0 - jnp.abs(jjf - sycol), 0.0).astype(jnp.bfloat16)
            rx = jnp.maximum(1.0 - jnp.abs(kkf - sxcol), 0.0).astype(jnp.bfloat16)
            for c in range(C):
                im = imgs_ref[b, c]
                p = lax.dot_general(
                    im, rx, (((1,), (1,)), ((), ())),
                    preferred_element_type=jnp.float32).astype(jnp.bfloat16)
                big = lax.dot_general(
                    ry, p, (((1,), (0,)), ((), ())),
                    preferred_element_type=jnp.float32)
                for g in range(G):
                    outc = big[g * OUT_SIZE:(g + 1) * OUT_SIZE,
                               g * OUT_SIZE:(g + 1) * OUT_SIZE]
                    crops_ref[b, c, pl.ds(k * G + g, 1), :, :] = (
                        outc.reshape(1, OUT_SIZE, OUT_SIZE))
        return 0

    lax.fori_loop(0, NCHUNK, chunk, 0)


def kernel(rects, img):
    B, N, _ = rects.shape
    _, H, W, C = img.shape
    R8 = 8
    CN = N // R8
    rects_t = rects.transpose(0, 2, 1).reshape(B, 5, R8, CN)
    cpad = (-CN) % 8
    coordt = rects[:, :, :4].reshape(B, R8, CN, 4).transpose(0, 3, 2, 1)
    coordt = jnp.pad(coordt, ((0, 0), (0, 0), (0, cpad), (0, 0)))
    img2 = img.reshape(B, H, W * C).astype(jnp.bfloat16)
    lidx = jnp.arange(W * C, dtype=jnp.int32)[None, :, None]
    xidx = jnp.arange(W, dtype=jnp.int32)[None, None, :]
    cidx = jnp.arange(C, dtype=jnp.int32)[:, None, None]
    sel = (lidx == C * xidx + cidx).astype(jnp.bfloat16)
    crops_t, bb8 = pl.pallas_call(
        _body,
        grid=(1,),
        in_specs=[
            pl.BlockSpec((B, 5, R8, CN), lambda b: (0, 0, 0, 0)),
            pl.BlockSpec((B, 4, CN + cpad, R8), lambda b: (0, 0, 0, 0)),
            pl.BlockSpec((B, H, W * C), lambda b: (0, 0, 0)),
            pl.BlockSpec((C, W * C, W), lambda b: (0, 0, 0)),
        ],
        out_specs=[
            pl.BlockSpec((B, C, MAX_OUT, OUT_SIZE, OUT_SIZE),
                         lambda b: (0, 0, 0, 0, 0)),
            pl.BlockSpec((B, MAX_OUT, 8), lambda b: (0, 0, 0)),
        ],
        out_shape=[
            jax.ShapeDtypeStruct((B, C, MAX_OUT, OUT_SIZE, OUT_SIZE),
                                 jnp.float32),
            jax.ShapeDtypeStruct((B, MAX_OUT, 8), jnp.float32),
        ],
        scratch_shapes=[pltpu.SMEM((B, MAX_OUT, 8), jnp.float32),
                        pltpu.VMEM((B, R8, CN), jnp.float32),
                        pltpu.VMEM((B, C, H, W), jnp.bfloat16)],
    )(rects_t, coordt, img2, sel)
    crops = crops_t.transpose(0, 2, 3, 4, 1)
    bb = bb8[..., :5]
    return crops, bb


# speedup vs baseline: 1.4231x; 1.4231x over previous
import jax
import jax.numpy as jnp
from jax import lax
from jax.experimental import pallas as pl
from jax.experimental.pallas import tpu as pltpu

MAX_OUT = 100
IOU_THR = 0.7
OUT_SIZE = 24
NEG = float("-inf")
G = 10
NCHUNK = MAX_OUT // G


def _body(rects_ref, coordt_ref, img_ref, crops_ref, bb_ref, sm_ref, area_ref):
    B = rects_ref.shape[0]
    R8, CN = rects_ref.shape[2], rects_ref.shape[3]
    H, W = img_ref.shape[2], img_ref.shape[3]

    flat = (lax.broadcasted_iota(jnp.int32, (R8, CN), 0) * CN
            + lax.broadcasted_iota(jnp.int32, (R8, CN), 1))
    bigint = jnp.int32(2 ** 30)
    subl8 = lax.broadcasted_iota(jnp.int32, (8, 8), 0)
    lane8g = lax.broadcasted_iota(jnp.int32, (8, 8), 1)
    lane8 = lax.broadcasted_iota(jnp.int32, (1, 8), 1)

    for b in range(B):
        area_ref[b] = ((rects_ref[b, 2] - rects_ref[b, 0])
                       * (rects_ref[b, 3] - rects_ref[b, 1]))

    def step(i, carry):
        ms = [jnp.max(carry[b]) for b in range(B)]
        idxs = [jnp.min(jnp.where(carry[b] == ms[b], flat, bigint))
                for b in range(B)]
        outs = []
        for b in range(B):
            m, idxsel = ms[b], idxs[b]
            valid = m > NEG
            cc = idxsel % CN
            rr = idxsel // CN
            ca = pl.multiple_of((cc // 8) * 8, 8)
            ss = cc - ca
            pick = (subl8 == ss) & (lane8g == rr)
            bx1 = jnp.sum(jnp.where(pick, coordt_ref[b, 0, pl.ds(ca, 8), :], 0.0))
            by1 = jnp.sum(jnp.where(pick, coordt_ref[b, 1, pl.ds(ca, 8), :], 0.0))
            bx2 = jnp.sum(jnp.where(pick, coordt_ref[b, 2, pl.ds(ca, 8), :], 0.0))
            by2 = jnp.sum(jnp.where(pick, coordt_ref[b, 3, pl.ds(ca, 8), :], 0.0))

            x1a = rects_ref[b, 0]
            y1a = rects_ref[b, 1]
            x2a = rects_ref[b, 2]
            y2a = rects_ref[b, 3]
            ix1 = jnp.maximum(bx1, x1a)
            iy1 = jnp.maximum(by1, y1a)
            ix2 = jnp.minimum(bx2, x2a)
            iy2 = jnp.minimum(by2, y2a)
            inter = (jnp.maximum(ix2 - ix1, 0.0)
                     * jnp.maximum(iy2 - iy1, 0.0))
            area_b = (bx2 - bx1) * (by2 - by1)
            iou = inter / (area_b + area_ref[b] - inter)
            supp = (iou > IOU_THR) & valid
            outs.append(jnp.where(supp, NEG, carry[b]))

            vx1 = jnp.where(valid, bx1, 0.0)
            vy1 = jnp.where(valid, by1, 0.0)
            vx2 = jnp.where(valid, bx2, 0.0)
            vy2 = jnp.where(valid, by2, 0.0)
            vm = jnp.where(valid, m, 0.0)
            row = (jnp.where(lane8 == 0, vx1, 0.0)
                   + jnp.where(lane8 == 1, vy1, 0.0)
                   + jnp.where(lane8 == 2, vx2, 0.0)
                   + jnp.where(lane8 == 3, vy2, 0.0)
                   + jnp.where(lane8 == 4, vm, 0.0))
            bb_ref[b, pl.ds(i, 1), :] = row[:, 0:5]
            sm_ref[b, i, 0] = vx1
            sm_ref[b, i, 1] = vy1
            sm_ref[b, i, 2] = vx2
            sm_ref[b, i, 3] = vy2
        return tuple(outs)

    lax.fori_loop(0, MAX_OUT, step,
                  tuple(rects_ref[b, 4] for b in range(B)))

    iic = lax.broadcasted_iota(jnp.int32, (OUT_SIZE, 1), 0).astype(jnp.float32)
    jjf = lax.broadcasted_iota(jnp.int32, (G * OUT_SIZE, H), 1).astype(jnp.float32)
    kkf = lax.broadcasted_iota(jnp.int32, (G * OUT_SIZE, W), 1).astype(jnp.float32)

    def chunk(k, _):
        rys, rxs = [], []
        for b in range(B):
            sy_l, sx_l = [], []
            for g in range(G):
                idx = k * G + g
                x1q = sm_ref[b, idx, 0].astype(jnp.int32)
                y1q = sm_ref[b, idx, 1].astype(jnp.int32)
                x2q = sm_ref[b, idx, 2].astype(jnp.int32)
                y2q = sm_ref[b, idx, 3].astype(jnp.int32)
                h = (y2q - y1q).astype(jnp.float32)
                w = (x2q - x1q).astype(jnp.float32)
                y0 = (y1q - 1).astype(jnp.float32)
                x0 = (x1q - 1).astype(jnp.float32)
                sy_l.append(jnp.maximum(y0 + iic * h / OUT_SIZE, 0.0))
                sx_l.append(jnp.maximum(x0 + iic * w / OUT_SIZE, 0.0))
            sycol = jnp.concatenate(sy_l, axis=0)
            sxcol = jnp.concatenate(sx_l, axis=0)
            rys.append(jnp.maximum(1.0 - jnp.abs(jjf - sycol),
                                   0.0).astype(jnp.bfloat16))
            rxs.append(jnp.maximum(1.0 - jnp.abs(kkf - sxcol),
                                   0.0).astype(jnp.bfloat16))
        ps = []
        for b in range(B):
            for c in range(3):
                ps.append(lax.dot_general(
                    img_ref[b, c], rxs[b], (((1,), (1,)), ((), ())),
                    preferred_element_type=jnp.float32).astype(jnp.bfloat16))
        for b in range(B):
            for c in range(3):
                big = lax.dot_general(
                    rys[b], ps[b * 3 + c], (((1,), (0,)), ((), ())),
                    preferred_element_type=jnp.float32)
                for g in range(G):
                    outc = big[g * OUT_SIZE:(g + 1) * OUT_SIZE,
                               g * OUT_SIZE:(g + 1) * OUT_SIZE]
                    crops_ref[b, c, pl.ds(k * G + g, 1), :, :] = (
                        outc.reshape(1, OUT_SIZE, OUT_SIZE))
        return 0

    lax.fori_loop(0, NCHUNK, chunk, 0)


def kernel(rects, img):
    B, N, _ = rects.shape
    _, H, W, C = img.shape
    R8 = 8
    CN = N // R8
    rects_t = rects.transpose(0, 2, 1).reshape(B, 5, R8, CN)
    cpad = (-CN) % 8
    coordt = rects[:, :, :4].reshape(B, R8, CN, 4).transpose(0, 3, 2, 1)
    coordt = jnp.pad(coordt, ((0, 0), (0, 0), (0, cpad), (0, 0)))
    img_t = img.transpose(0, 3, 1, 2).astype(jnp.bfloat16)
    crops_t, bb = pl.pallas_call(
        _body,
        grid=(1,),
        in_specs=[
            pl.BlockSpec((B, 5, R8, CN), lambda b: (0, 0, 0, 0)),
            pl.BlockSpec((B, 4, CN + cpad, R8), lambda b: (0, 0, 0, 0)),
            pl.BlockSpec((B, C, H, W), lambda b: (0, 0, 0, 0)),
        ],
        out_specs=[
            pl.BlockSpec((B, C, MAX_OUT, OUT_SIZE, OUT_SIZE),
                         lambda b: (0, 0, 0, 0, 0)),
            pl.BlockSpec((B, MAX_OUT, 5), lambda b: (0, 0, 0)),
        ],
        out_shape=[
            jax.ShapeDtypeStruct((B, C, MAX_OUT, OUT_SIZE, OUT_SIZE),
                                 jnp.float32),
            jax.ShapeDtypeStruct((B, MAX_OUT, 5), jnp.float32),
        ],
        scratch_shapes=[pltpu.SMEM((B, MAX_OUT, 8), jnp.float32),
                        pltpu.VMEM((B, R8, CN), jnp.float32)],
    )(rects_t, coordt, img_t)
    crops = crops_t.transpose(0, 2, 3, 4, 1)
    return crops, bb
